# baseline (speedup 1.0000x reference)
import jax
import jax.numpy as jnp
from jax.experimental import pallas as pl
from jax.experimental.pallas import tpu as pltpu

B, N, DIM = 1, 2048, 1024
H, DH, BS = 16, 64, 64
W = N // BS
SEL = 8
NMEM = 1
HID = BS * DH
SCALE = DH ** -0.5
NEG = -1e30
WP = 40
QT = 256


_PRECISION = jax.lax.Precision.DEFAULT


def _f32dot(a, b):
    return jnp.dot(a, b, preferred_element_type=jnp.float32,
                   precision=_PRECISION)


def _cdot(a, b):
    return jax.lax.dot_general(a, b, (((1,), (1,)), ((), ())),
                               preferred_element_type=jnp.float32,
                               precision=_PRECISION)


def _proj_kernel(x_ref, wq_ref, wk_ref, wv_ref, nw_ref, gw_ref, gb_ref,
                 q_ref, k_ref, v_ref, g_ref):
    x = x_ref[...]
    q_ref[...] = _f32dot(x, wq_ref[...])
    k_ref[...] = _f32dot(x, wk_ref[...])
    v_ref[...] = _f32dot(x, wv_ref[...])
    ms = jnp.mean(x * x, axis=1, keepdims=True)
    xn = x * jax.lax.rsqrt(ms + 1e-6) * nw_ref[...]
    g_ref[...] = jax.nn.sigmoid(_f32dot(xn, gw_ref[...]) + gb_ref[...])


def _proj(x, Wq, Wk, Wv, norm_w, gate_w, gate_b):
    grid = (N // QT,)
    row = pl.BlockSpec((QT, DIM), lambda i: (i, 0))
    full = pl.BlockSpec((DIM, DIM), lambda i: (0, 0))
    return pl.pallas_call(
        _proj_kernel,
        grid=grid,
        in_specs=[row, full, full, full,
                  pl.BlockSpec((1, DIM), lambda i: (0, 0)),
                  pl.BlockSpec((DIM, 2 * H), lambda i: (0, 0)),
                  pl.BlockSpec((1, 2 * H), lambda i: (0, 0))],
        out_specs=[row, row, row,
                   pl.BlockSpec((QT, 2 * H), lambda i: (i, 0))],
        out_shape=[jax.ShapeDtypeStruct((N, DIM), jnp.float32)] * 3 +
                  [jax.ShapeDtypeStruct((N, 2 * H), jnp.float32)],
    )(x, Wq, Wk, Wv, norm_w[None, :], gate_w, gate_b[None, :])


def _compress_kernel(a_ref, p_ref, w1_ref, b1_ref, w2_ref, b2_ref, out_ref):
    j = pl.program_id(0)
    a = a_ref[...] + p_ref[...]
    h1 = jax.nn.relu(_f32dot(a, w1_ref[...]) + b1_ref[...])

    @pl.when(j == 0)
    def _():
        out_ref[...] = jnp.broadcast_to(b2_ref[...], (H * W, DH))

    out_ref[...] += _f32dot(h1, w2_ref[...])


def _compress(a, p, w1, b1, w2, b2, bn=512):
    grid = (HID // bn,)
    return pl.pallas_call(
        _compress_kernel,
        grid=grid,
        in_specs=[pl.BlockSpec((H * W, HID), lambda j: (0, 0)),
                  pl.BlockSpec((H * W, HID), lambda j: (0, 0)),
                  pl.BlockSpec((HID, bn), lambda j: (0, j)),
                  pl.BlockSpec((1, bn), lambda j: (0, j)),
                  pl.BlockSpec((bn, DH), lambda j: (j, 0)),
                  pl.BlockSpec((1, DH), lambda j: (0, 0))],
        out_specs=pl.BlockSpec((H * W, DH), lambda j: (0, 0)),
        out_shape=jax.ShapeDtypeStruct((H * W, DH), jnp.float32),
    )(a, p, w1, b1[None, :], w2, b2[None, :])


def _rope(x, cos, sin):
    x1 = x[:, :DH // 2]
    x2 = x[:, DH // 2:]
    rot = jnp.concatenate([-x2, x1], axis=1)
    return x * cos + rot * sin


KC = 512
NKC = N // KC


def _attn_kernel(q_ref, k_ref, v_ref, ckf_ref, cvf_ref, cos_ref, sin_ref,
                 cout_ref, fout_ref, fk_ref):
    qt = pl.program_id(1)

    @pl.when(qt == 0)
    def _():
        fk_ref[...] = _rope(k_ref[0], cos_ref[...], sin_ref[...])

    q = q_ref[0]

    sc = _cdot(ckf_ref[0], q) * SCALE
    c = jax.lax.broadcasted_iota(jnp.int32, (WP, QT), 0)
    iq = qt * QT + jax.lax.broadcasted_iota(jnp.int32, (WP, QT), 1)
    valid = (c == 0) | ((c <= W) & (iq >= c * BS - 1))
    sc = jnp.where(valid, sc, NEG)
    mc = jnp.max(sc, axis=0, keepdims=True)
    ec = jnp.exp(sc - mc)
    attnc = ec / jnp.sum(ec, axis=0, keepdims=True)
    cout_ref[0] = jax.lax.dot_general(
        attnc, cvf_ref[0], (((0,), (0,)), ((), ())),
        preferred_element_type=jnp.float32, precision=_PRECISION)
    blkrow = (c >= 1) & (c <= W)
    imp = jnp.where(blkrow, attnc, -1.0)
    vt = imp
    thr = vt
    for _ in range(SEL):
        thr = jnp.max(vt, axis=0, keepdims=True)
        vt = jnp.where(vt == thr, -1.0, vt)
    selT = (imp >= thr).astype(jnp.float32)

    last = qt * QT // KC
    cs = cos_ref[pl.ds(qt * QT, QT), :]
    sn = sin_ref[pl.ds(qt * QT, QT), :]
    fq = _rope(q, cs, sn) * SCALE

    crow = jax.lax.broadcasted_iota(jnp.int32, (WP, KC), 0)
    pcol = jax.lax.broadcasted_iota(jnp.int32, (WP, KC), 1) // BS + NMEM

    def selmask(c):
        expand = (crow == pcol + c * (KC // BS)).astype(jnp.float32)
        return jax.lax.dot_general(
            selT, expand, (((0,), (0,)), ((), ())),
            preferred_element_type=jnp.float32, precision=_PRECISION) > 0.5

    def step(m, l, acc, c, allowed, s):
        s = jnp.where(allowed, s, NEG)
        m_new = jnp.maximum(m, jnp.max(s, axis=1, keepdims=True))
        e = jnp.where(allowed, jnp.exp(s - m_new), 0.0)
        corr = jnp.exp(m - m_new)
        l_new = l * corr + jnp.sum(e, axis=1, keepdims=True)
        acc_new = acc * corr + _f32dot(e, v_ref[0, pl.ds(c * KC, KC), :])
        return m_new, l_new, acc_new

    def interior(cc, carry):
        m, l, acc = carry
        s = _cdot(fq, fk_ref[pl.ds(cc * KC, KC), :])
        return step(m, l, acc, cc, selmask(cc), s)

    m0 = jnp.full((QT, 1), NEG, jnp.float32)
    l0 = jnp.zeros((QT, 1), jnp.float32)
    a0 = jnp.zeros((QT, DH), jnp.float32)
    m, l, acc = jax.lax.fori_loop(0, last, interior, (m0, l0, a0))

    s = _cdot(fq, fk_ref[pl.ds(last * KC, KC), :])
    i = qt * QT + jax.lax.broadcasted_iota(jnp.int32, (QT, KC), 0)
    p = last * KC + jax.lax.broadcasted_iota(jnp.int32, (QT, KC), 1)
    allowed = (p <= i) & (selmask(last) | ((p // BS) == (i // BS)))
    m, l, acc = step(m, l, acc, last, allowed, s)
    fout_ref[0] = acc / l


def _attn(q, k, v, ckf, cvf, cos, sin):
    qtile = pl.BlockSpec((1, QT, DH), lambda h, t: (h, t, 0))
    head = pl.BlockSpec((1, N, DH), lambda h, t: (h, 0, 0))
    cf = pl.BlockSpec((1, WP, DH), lambda h, t: (h, 0, 0))
    full = pl.BlockSpec((N, DH), lambda h, t: (0, 0))
    return pl.pallas_call(
        _attn_kernel,
        grid=(H, N // QT),
        in_specs=[qtile, head, head, cf, cf, full, full],
        out_specs=[qtile, qtile],
        out_shape=[jax.ShapeDtypeStruct((H, N, DH), jnp.float32)] * 2,
        scratch_shapes=[pltpu.VMEM((N, DH), jnp.float32)],
    )(q, k, v, ckf, cvf, cos, sin)


def _final_kernel(c_ref, f_ref, gc_ref, gf_ref, wo_ref, out_ref):
    hrow = jax.lax.broadcasted_iota(jnp.int32, (H, DIM), 0)
    dcol = jax.lax.broadcasted_iota(jnp.int32, (H, DIM), 1)
    expand = (hrow == dcol // DH).astype(jnp.float32)
    gc = jnp.dot(gc_ref[...], expand, preferred_element_type=jnp.float32,
                 precision=jax.lax.Precision.HIGHEST)
    gf = jnp.dot(gf_ref[...], expand, preferred_element_type=jnp.float32,
                 precision=jax.lax.Precision.HIGHEST)
    merged = gc * c_ref[...] + gf * f_ref[...]
    out_ref[...] = _f32dot(merged, wo_ref[...])


def _final(c, f, gc, gf, Wo):
    row = pl.BlockSpec((QT, DIM), lambda i: (i, 0))
    return pl.pallas_call(
        _final_kernel,
        grid=(N // QT,),
        in_specs=[row, row,
                  pl.BlockSpec((QT, H), lambda i: (i, 0)),
                  pl.BlockSpec((QT, H), lambda i: (i, 0)),
                  pl.BlockSpec((DIM, DIM), lambda i: (0, 0))],
        out_specs=row,
        out_shape=jax.ShapeDtypeStruct((N, DIM), jnp.float32),
    )(c, f, gc, gf, Wo)


def kernel(inp, Wq, Wk, Wv, norm_w, mem_kv, k_pos, v_pos,
           kc_w1, kc_b1, kc_w2, kc_b2, vc_w1, vc_b1, vc_w2, vc_b2,
           gate_w, gate_b, Wo):
    x = inp[0]
    q, k, v, g = _proj(x, Wq, Wk, Wv, norm_w, gate_w, gate_b)

    def to_blocks(t):
        return (t.reshape(W, BS, H, DH).transpose(2, 0, 1, 3)
                .reshape(H * W, HID))

    ck = _compress(to_blocks(k), jnp.repeat(k_pos.reshape(H, HID), W, axis=0),
                   kc_w1, kc_b1, kc_w2, kc_b2)
    cv = _compress(to_blocks(v), jnp.repeat(v_pos.reshape(H, HID), W, axis=0),
                   vc_w1, vc_b1, vc_w2, vc_b2)
    ckf = jnp.concatenate([mem_kv[0], ck.reshape(H, W, DH)], axis=1)
    cvf = jnp.concatenate([mem_kv[1], cv.reshape(H, W, DH)], axis=1)
    pad = ((0, 0), (0, WP - W - NMEM), (0, 0))
    ckf = jnp.pad(ckf, pad)
    cvf = jnp.pad(cvf, pad)

    pos = jnp.arange(N, dtype=jnp.float32)
    inv = 1.0 / (10000.0 ** (jnp.arange(0, DH, 2, dtype=jnp.float32) / DH))
    f = pos[:, None] * inv[None, :]
    emb = jnp.concatenate([f, f], axis=1)
    cos = jnp.cos(emb)
    sin = jnp.sin(emb)

    def to_heads(t):
        return t.reshape(N, H, DH).transpose(1, 0, 2)

    cout, fout = _attn(to_heads(q), to_heads(k), to_heads(v),
                       ckf, cvf, cos, sin)

    def from_heads(t):
        return t.transpose(1, 0, 2).reshape(N, DIM)

    out = _final(from_heads(cout), from_heads(fout),
                 g[:, 0::2], g[:, 1::2], Wo)
    return out[None]


# speedup vs baseline: 1.0068x; 1.0068x over previous
import jax
import jax.numpy as jnp
from jax.experimental import pallas as pl
from jax.experimental.pallas import tpu as pltpu

B, N, DIM = 1, 2048, 1024
H, DH, BS = 16, 64, 64
W = N // BS
SEL = 8
NMEM = 1
HID = BS * DH
SCALE = DH ** -0.5
NEG = -1e30
WP = 40
QT = 256


_PRECISION = jax.lax.Precision.DEFAULT


def _f32dot(a, b):
    return jnp.dot(a, b, preferred_element_type=jnp.float32,
                   precision=_PRECISION)


def _cdot(a, b):
    return jax.lax.dot_general(a, b, (((1,), (1,)), ((), ())),
                               preferred_element_type=jnp.float32,
                               precision=_PRECISION)


def _proj_kernel(x_ref, wq_ref, wk_ref, wv_ref, nw_ref, gw_ref, gb_ref,
                 q_ref, k_ref, v_ref, g_ref):
    x = x_ref[...]
    q_ref[...] = _f32dot(x, wq_ref[...])
    k_ref[...] = _f32dot(x, wk_ref[...])
    v_ref[...] = _f32dot(x, wv_ref[...])
    ms = jnp.mean(x * x, axis=1, keepdims=True)
    xn = x * jax.lax.rsqrt(ms + 1e-6) * nw_ref[...]
    g_ref[...] = jax.nn.sigmoid(_f32dot(xn, gw_ref[...]) + gb_ref[...])


def _proj(x, Wq, Wk, Wv, norm_w, gate_w, gate_b):
    grid = (N // QT,)
    row = pl.BlockSpec((QT, DIM), lambda i: (i, 0))
    full = pl.BlockSpec((DIM, DIM), lambda i: (0, 0))
    return pl.pallas_call(
        _proj_kernel,
        grid=grid,
        in_specs=[row, full, full, full,
                  pl.BlockSpec((1, DIM), lambda i: (0, 0)),
                  pl.BlockSpec((DIM, 2 * H), lambda i: (0, 0)),
                  pl.BlockSpec((1, 2 * H), lambda i: (0, 0))],
        out_specs=[row, row, row,
                   pl.BlockSpec((QT, 2 * H), lambda i: (i, 0))],
        out_shape=[jax.ShapeDtypeStruct((N, DIM), jnp.float32)] * 3 +
                  [jax.ShapeDtypeStruct((N, 2 * H), jnp.float32)],
    )(x, Wq, Wk, Wv, norm_w[None, :], gate_w, gate_b[None, :])


def _compress_kernel(a_ref, p_ref, w1_ref, b1_ref, w2_ref, b2_ref, out_ref):
    j = pl.program_id(0)
    a = a_ref[...] + p_ref[...]
    h1 = jax.nn.relu(_f32dot(a, w1_ref[...]) + b1_ref[...])

    @pl.when(j == 0)
    def _():
        out_ref[...] = jnp.broadcast_to(b2_ref[...], (H * W, DH))

    out_ref[...] += _f32dot(h1, w2_ref[...])


def _compress(a, p, w1, b1, w2, b2, bn=512):
    grid = (HID // bn,)
    return pl.pallas_call(
        _compress_kernel,
        grid=grid,
        in_specs=[pl.BlockSpec((H * W, HID), lambda j: (0, 0)),
                  pl.BlockSpec((H * W, HID), lambda j: (0, 0)),
                  pl.BlockSpec((HID, bn), lambda j: (0, j)),
                  pl.BlockSpec((1, bn), lambda j: (0, j)),
                  pl.BlockSpec((bn, DH), lambda j: (j, 0)),
                  pl.BlockSpec((1, DH), lambda j: (0, 0))],
        out_specs=pl.BlockSpec((H * W, DH), lambda j: (0, 0)),
        out_shape=jax.ShapeDtypeStruct((H * W, DH), jnp.float32),
    )(a, p, w1, b1[None, :], w2, b2[None, :])


def _rope(x, cos, sin):
    x1 = x[:, :DH // 2]
    x2 = x[:, DH // 2:]
    rot = jnp.concatenate([-x2, x1], axis=1)
    return x * cos + rot * sin


KC = 512
NKC = N // KC


def _coarse_kernel(q_ref, ckf_ref, cvf_ref, cout_ref, selm_ref):
    q = q_ref[0]
    sc = _cdot(ckf_ref[0], q) * SCALE
    c = jax.lax.broadcasted_iota(jnp.int32, (WP, N), 0)
    iq = jax.lax.broadcasted_iota(jnp.int32, (WP, N), 1)
    valid = (c == 0) | ((c <= W) & (iq >= c * BS - 1))
    sc = jnp.where(valid, sc, NEG)
    mc = jnp.max(sc, axis=0, keepdims=True)
    ec = jnp.exp(sc - mc)
    attnc = ec / jnp.sum(ec, axis=0, keepdims=True)
    cout_ref[0] = jax.lax.dot_general(
        attnc, cvf_ref[0], (((0,), (0,)), ((), ())),
        preferred_element_type=jnp.float32, precision=_PRECISION)
    blkrow = (c >= 1) & (c <= W)
    imp = jnp.where(blkrow, attnc, -1.0)
    vt = imp
    thr = vt
    for _ in range(SEL):
        thr = jnp.max(vt, axis=0, keepdims=True)
        vt = jnp.where(vt == thr, -1.0, vt)
    selm_ref[0] = (imp >= thr).astype(jnp.float32)


def _coarse(q, ckf, cvf):
    head = pl.BlockSpec((1, N, DH), lambda h: (h, 0, 0))
    cf = pl.BlockSpec((1, WP, DH), lambda h: (h, 0, 0))
    return pl.pallas_call(
        _coarse_kernel,
        grid=(H,),
        in_specs=[head, cf, cf],
        out_specs=[head, pl.BlockSpec((1, WP, N), lambda h: (h, 0, 0))],
        out_shape=[jax.ShapeDtypeStruct((H, N, DH), jnp.float32),
                   jax.ShapeDtypeStruct((H, WP, N), jnp.float32)],
    )(q, ckf, cvf)


def _fine_kernel(q_ref, k_ref, v_ref, sel_ref, cos_ref, sin_ref,
                 fout_ref, fk_ref):
    qt = pl.program_id(1)

    @pl.when(qt == 0)
    def _():
        fk_ref[...] = _rope(k_ref[0], cos_ref[...], sin_ref[...])

    last = qt * QT // KC
    cs = cos_ref[pl.ds(qt * QT, QT), :]
    sn = sin_ref[pl.ds(qt * QT, QT), :]
    fq = _rope(q_ref[0], cs, sn) * SCALE
    selT = sel_ref[0]

    crow = jax.lax.broadcasted_iota(jnp.int32, (WP, KC), 0)
    pcol = jax.lax.broadcasted_iota(jnp.int32, (WP, KC), 1) // BS + NMEM

    def selmask(c):
        expand = (crow == pcol + c * (KC // BS)).astype(jnp.float32)
        return jax.lax.dot_general(
            selT, expand, (((0,), (0,)), ((), ())),
            preferred_element_type=jnp.float32, precision=_PRECISION) > 0.5

    def step(m, l, acc, c, allowed, s):
        s = jnp.where(allowed, s, NEG)
        m_new = jnp.maximum(m, jnp.max(s, axis=1, keepdims=True))
        e = jnp.where(allowed, jnp.exp(s - m_new), 0.0)
        corr = jnp.exp(m - m_new)
        l_new = l * corr + jnp.sum(e, axis=1, keepdims=True)
        acc_new = acc * corr + _f32dot(e, v_ref[0, pl.ds(c * KC, KC), :])
        return m_new, l_new, acc_new

    def interior(cc, carry):
        m, l, acc = carry
        s = _cdot(fq, fk_ref[pl.ds(cc * KC, KC), :])
        return step(m, l, acc, cc, selmask(cc), s)

    m0 = jnp.full((QT, 1), NEG, jnp.float32)
    l0 = jnp.zeros((QT, 1), jnp.float32)
    a0 = jnp.zeros((QT, DH), jnp.float32)
    m, l, acc = jax.lax.fori_loop(0, last, interior, (m0, l0, a0))

    s = _cdot(fq, fk_ref[pl.ds(last * KC, KC), :])
    i = qt * QT + jax.lax.broadcasted_iota(jnp.int32, (QT, KC), 0)
    p = last * KC + jax.lax.broadcasted_iota(jnp.int32, (QT, KC), 1)
    allowed = (p <= i) & (selmask(last) | ((p // BS) == (i // BS)))
    m, l, acc = step(m, l, acc, last, allowed, s)
    fout_ref[0] = acc / l


def _fine(q, k, v, selm, cos, sin):
    qtile = pl.BlockSpec((1, QT, DH), lambda h, t: (h, t, 0))
    head = pl.BlockSpec((1, N, DH), lambda h, t: (h, 0, 0))
    full = pl.BlockSpec((N, DH), lambda h, t: (0, 0))
    return pl.pallas_call(
        _fine_kernel,
        grid=(H, N // QT),
        in_specs=[qtile, head, head,
                  pl.BlockSpec((1, WP, QT), lambda h, t: (h, 0, t)),
                  full, full],
        out_specs=qtile,
        out_shape=jax.ShapeDtypeStruct((H, N, DH), jnp.float32),
        scratch_shapes=[pltpu.VMEM((N, DH), jnp.float32)],
    )(q, k, v, selm, cos, sin)


def _final_kernel(c_ref, f_ref, gc_ref, gf_ref, wo_ref, out_ref):
    hrow = jax.lax.broadcasted_iota(jnp.int32, (H, DIM), 0)
    dcol = jax.lax.broadcasted_iota(jnp.int32, (H, DIM), 1)
    expand = (hrow == dcol // DH).astype(jnp.float32)
    gc = jnp.dot(gc_ref[...], expand, preferred_element_type=jnp.float32,
                 precision=jax.lax.Precision.HIGHEST)
    gf = jnp.dot(gf_ref[...], expand, preferred_element_type=jnp.float32,
                 precision=jax.lax.Precision.HIGHEST)
    merged = gc * c_ref[...] + gf * f_ref[...]
    out_ref[...] = _f32dot(merged, wo_ref[...])


def _final(c, f, gc, gf, Wo):
    row = pl.BlockSpec((QT, DIM), lambda i: (i, 0))
    return pl.pallas_call(
        _final_kernel,
        grid=(N // QT,),
        in_specs=[row, row,
                  pl.BlockSpec((QT, H), lambda i: (i, 0)),
                  pl.BlockSpec((QT, H), lambda i: (i, 0)),
                  pl.BlockSpec((DIM, DIM), lambda i: (0, 0))],
        out_specs=row,
        out_shape=jax.ShapeDtypeStruct((N, DIM), jnp.float32),
    )(c, f, gc, gf, Wo)


def kernel(inp, Wq, Wk, Wv, norm_w, mem_kv, k_pos, v_pos,
           kc_w1, kc_b1, kc_w2, kc_b2, vc_w1, vc_b1, vc_w2, vc_b2,
           gate_w, gate_b, Wo):
    x = inp[0]
    q, k, v, g = _proj(x, Wq, Wk, Wv, norm_w, gate_w, gate_b)

    def to_blocks(t):
        return (t.reshape(W, BS, H, DH).transpose(2, 0, 1, 3)
                .reshape(H * W, HID))

    ck = _compress(to_blocks(k), jnp.repeat(k_pos.reshape(H, HID), W, axis=0),
                   kc_w1, kc_b1, kc_w2, kc_b2)
    cv = _compress(to_blocks(v), jnp.repeat(v_pos.reshape(H, HID), W, axis=0),
                   vc_w1, vc_b1, vc_w2, vc_b2)
    ckf = jnp.concatenate([mem_kv[0], ck.reshape(H, W, DH)], axis=1)
    cvf = jnp.concatenate([mem_kv[1], cv.reshape(H, W, DH)], axis=1)
    pad = ((0, 0), (0, WP - W - NMEM), (0, 0))
    ckf = jnp.pad(ckf, pad)
    cvf = jnp.pad(cvf, pad)

    pos = jnp.arange(N, dtype=jnp.float32)
    inv = 1.0 / (10000.0 ** (jnp.arange(0, DH, 2, dtype=jnp.float32) / DH))
    f = pos[:, None] * inv[None, :]
    emb = jnp.concatenate([f, f], axis=1)
    cos = jnp.cos(emb)
    sin = jnp.sin(emb)

    def to_heads(t):
        return t.reshape(N, H, DH).transpose(1, 0, 2)

    qh, kh, vh = to_heads(q), to_heads(k), to_heads(v)
    cout, selm = _coarse(qh, ckf, cvf)
    fout = _fine(qh, kh, vh, selm, cos, sin)

    def from_heads(t):
        return t.transpose(1, 0, 2).reshape(N, DIM)

    out = _final(from_heads(cout), from_heads(fout),
                 g[:, 0::2], g[:, 1::2], Wo)
    return out[None]


# speedup vs baseline: 1.0435x; 1.0364x over previous
import jax
import jax.numpy as jnp
from jax.experimental import pallas as pl
from jax.experimental.pallas import tpu as pltpu

B, N, DIM = 1, 2048, 1024
H, DH, BS = 16, 64, 64
W = N // BS
SEL = 8
NMEM = 1
HID = BS * DH
SCALE = DH ** -0.5
NEG = -1e30
WP = 40
QT = 256


_PRECISION = jax.lax.Precision.DEFAULT


def _f32dot(a, b):
    return jnp.dot(a, b, preferred_element_type=jnp.float32,
                   precision=_PRECISION)


def _cdot(a, b):
    return jax.lax.dot_general(a, b, (((1,), (1,)), ((), ())),
                               preferred_element_type=jnp.float32,
                               precision=_PRECISION)


def _proj_kernel(x_ref, wq_ref, wk_ref, wv_ref, nw_ref, gw_ref, gb_ref,
                 q_ref, k_ref, v_ref, g_ref):
    x = x_ref[...]
    q_ref[...] = _f32dot(x, wq_ref[...])
    k_ref[...] = _f32dot(x, wk_ref[...])
    v_ref[...] = _f32dot(x, wv_ref[...])
    ms = jnp.mean(x * x, axis=1, keepdims=True)
    xn = x * jax.lax.rsqrt(ms + 1e-6) * nw_ref[...]
    g_ref[...] = jax.nn.sigmoid(_f32dot(xn, gw_ref[...]) + gb_ref[...])


def _proj(x, Wq, Wk, Wv, norm_w, gate_w, gate_b):
    grid = (N // QT,)
    row = pl.BlockSpec((QT, DIM), lambda i: (i, 0))
    full = pl.BlockSpec((DIM, DIM), lambda i: (0, 0))
    return pl.pallas_call(
        _proj_kernel,
        grid=grid,
        in_specs=[row, full, full, full,
                  pl.BlockSpec((1, DIM), lambda i: (0, 0)),
                  pl.BlockSpec((DIM, 2 * H), lambda i: (0, 0)),
                  pl.BlockSpec((1, 2 * H), lambda i: (0, 0))],
        out_specs=[row, row, row,
                   pl.BlockSpec((QT, 2 * H), lambda i: (i, 0))],
        out_shape=[jax.ShapeDtypeStruct((N, DIM), jnp.float32)] * 3 +
                  [jax.ShapeDtypeStruct((N, 2 * H), jnp.float32)],
    )(x, Wq, Wk, Wv, norm_w[None, :], gate_w, gate_b[None, :])


def _compress_kernel(a_ref, p_ref, w1_ref, b1_ref, w2_ref, b2_ref, out_ref):
    j = pl.program_id(0)
    a = a_ref[...] + p_ref[...]
    h1 = jax.nn.relu(_f32dot(a, w1_ref[...]) + b1_ref[...])

    @pl.when(j == 0)
    def _():
        out_ref[...] = jnp.broadcast_to(b2_ref[...], (H * W, DH))

    out_ref[...] += _f32dot(h1, w2_ref[...])


def _compress(a, p, w1, b1, w2, b2, bn=512):
    grid = (HID // bn,)
    return pl.pallas_call(
        _compress_kernel,
        grid=grid,
        in_specs=[pl.BlockSpec((H * W, HID), lambda j: (0, 0)),
                  pl.BlockSpec((H * W, HID), lambda j: (0, 0)),
                  pl.BlockSpec((HID, bn), lambda j: (0, j)),
                  pl.BlockSpec((1, bn), lambda j: (0, j)),
                  pl.BlockSpec((bn, DH), lambda j: (j, 0)),
                  pl.BlockSpec((1, DH), lambda j: (0, 0))],
        out_specs=pl.BlockSpec((H * W, DH), lambda j: (0, 0)),
        out_shape=jax.ShapeDtypeStruct((H * W, DH), jnp.float32),
    )(a, p, w1, b1[None, :], w2, b2[None, :])


def _rope(x, cos, sin):
    x1 = x[:, :DH // 2]
    x2 = x[:, DH // 2:]
    rot = jnp.concatenate([-x2, x1], axis=1)
    return x * cos + rot * sin


KC = 512
NKC = N // KC


def _coarse_kernel(q_ref, ckf_ref, cvf_ref, cout_ref, selm_ref):
    q = q_ref[0]
    sc = _cdot(ckf_ref[0], q) * SCALE
    c = jax.lax.broadcasted_iota(jnp.int32, (WP, N), 0)
    iq = jax.lax.broadcasted_iota(jnp.int32, (WP, N), 1)
    valid = (c == 0) | ((c <= W) & (iq >= c * BS - 1))
    sc = jnp.where(valid, sc, NEG)
    mc = jnp.max(sc, axis=0, keepdims=True)
    ec = jnp.exp(sc - mc)
    attnc = ec / jnp.sum(ec, axis=0, keepdims=True)
    cout_ref[0] = jax.lax.dot_general(
        attnc, cvf_ref[0], (((0,), (0,)), ((), ())),
        preferred_element_type=jnp.float32, precision=_PRECISION)
    blkrow = (c >= 1) & (c <= W)
    imp = jnp.where(blkrow, attnc, -1.0)
    vt = imp
    thr = vt
    for _ in range(SEL):
        thr = jnp.max(vt, axis=0, keepdims=True)
        vt = jnp.where(vt == thr, -1.0, vt)
    selm_ref[0] = (imp >= thr).astype(jnp.float32)


def _coarse(q, ckf, cvf):
    head = pl.BlockSpec((1, N, DH), lambda h: (h, 0, 0))
    cf = pl.BlockSpec((1, WP, DH), lambda h: (h, 0, 0))
    return pl.pallas_call(
        _coarse_kernel,
        grid=(H,),
        in_specs=[head, cf, cf],
        out_specs=[head, pl.BlockSpec((1, WP, N), lambda h: (h, 0, 0))],
        out_shape=[jax.ShapeDtypeStruct((H, N, DH), jnp.float32),
                   jax.ShapeDtypeStruct((H, WP, N), jnp.float32)],
    )(q, ckf, cvf)


def _fine_kernel(q_ref, k_ref, v_ref, sel_ref, cos_ref, sin_ref,
                 fout_ref, fk_ref):
    qt = pl.program_id(1)

    @pl.when(qt == 0)
    def _():
        fk_ref[...] = _rope(k_ref[0], cos_ref[...], sin_ref[...])

    last = qt * QT // KC
    cs = cos_ref[pl.ds(qt * QT, QT), :]
    sn = sin_ref[pl.ds(qt * QT, QT), :]
    fq = _rope(q_ref[0], cs, sn) * SCALE
    selT = sel_ref[0]

    crow = jax.lax.broadcasted_iota(jnp.int32, (WP, KC), 0)
    pcol = jax.lax.broadcasted_iota(jnp.int32, (WP, KC), 1) // BS + NMEM

    def selmask(c):
        expand = (crow == pcol + c * (KC // BS)).astype(jnp.float32)
        return jax.lax.dot_general(
            selT, expand, (((0,), (0,)), ((), ())),
            preferred_element_type=jnp.float32, precision=_PRECISION)

    def step(m, l, acc, c, maskf, s):
        m_new = jnp.maximum(m, jnp.max(s, axis=1, keepdims=True))
        e = jnp.exp(s - m_new) * maskf
        corr = jnp.exp(m - m_new)
        l_new = l * corr + jnp.sum(e, axis=1, keepdims=True)
        acc_new = acc * corr + _f32dot(e, v_ref[0, pl.ds(c * KC, KC), :])
        return m_new, l_new, acc_new

    def interior(cc, carry):
        m, l, acc = carry
        s = _cdot(fq, fk_ref[pl.ds(cc * KC, KC), :])
        return step(m, l, acc, cc, selmask(cc), s)

    m0 = jnp.full((QT, 1), NEG, jnp.float32)
    l0 = jnp.zeros((QT, 1), jnp.float32)
    a0 = jnp.zeros((QT, DH), jnp.float32)
    m, l, acc = jax.lax.fori_loop(0, last, interior, (m0, l0, a0))

    s = _cdot(fq, fk_ref[pl.ds(last * KC, KC), :])
    i = qt * QT + jax.lax.broadcasted_iota(jnp.int32, (QT, KC), 0)
    p = last * KC + jax.lax.broadcasted_iota(jnp.int32, (QT, KC), 1)
    own = ((p // BS) == (i // BS)).astype(jnp.float32)
    causal = (p <= i).astype(jnp.float32)
    sel = selmask(last)
    maskf = causal * (sel + own - sel * own)
    m, l, acc = step(m, l, acc, last, maskf, s)
    fout_ref[0] = acc / l


def _fine(q, k, v, selm, cos, sin):
    qtile = pl.BlockSpec((1, QT, DH), lambda h, t: (h, t, 0))
    head = pl.BlockSpec((1, N, DH), lambda h, t: (h, 0, 0))
    full = pl.BlockSpec((N, DH), lambda h, t: (0, 0))
    return pl.pallas_call(
        _fine_kernel,
        grid=(H, N // QT),
        in_specs=[qtile, head, head,
                  pl.BlockSpec((1, WP, QT), lambda h, t: (h, 0, t)),
                  full, full],
        out_specs=qtile,
        out_shape=jax.ShapeDtypeStruct((H, N, DH), jnp.float32),
        scratch_shapes=[pltpu.VMEM((N, DH), jnp.float32)],
    )(q, k, v, selm, cos, sin)


def _final_kernel(c_ref, f_ref, gc_ref, gf_ref, wo_ref, out_ref):
    hrow = jax.lax.broadcasted_iota(jnp.int32, (H, DIM), 0)
    dcol = jax.lax.broadcasted_iota(jnp.int32, (H, DIM), 1)
    expand = (hrow == dcol // DH).astype(jnp.float32)
    gc = jnp.dot(gc_ref[...], expand, preferred_element_type=jnp.float32,
                 precision=jax.lax.Precision.HIGHEST)
    gf = jnp.dot(gf_ref[...], expand, preferred_element_type=jnp.float32,
                 precision=jax.lax.Precision.HIGHEST)
    merged = gc * c_ref[...] + gf * f_ref[...]
    out_ref[...] = _f32dot(merged, wo_ref[...])


def _final(c, f, gc, gf, Wo):
    row = pl.BlockSpec((QT, DIM), lambda i: (i, 0))
    return pl.pallas_call(
        _final_kernel,
        grid=(N // QT,),
        in_specs=[row, row,
                  pl.BlockSpec((QT, H), lambda i: (i, 0)),
                  pl.BlockSpec((QT, H), lambda i: (i, 0)),
                  pl.BlockSpec((DIM, DIM), lambda i: (0, 0))],
        out_specs=row,
        out_shape=jax.ShapeDtypeStruct((N, DIM), jnp.float32),
    )(c, f, gc, gf, Wo)


def kernel(inp, Wq, Wk, Wv, norm_w, mem_kv, k_pos, v_pos,
           kc_w1, kc_b1, kc_w2, kc_b2, vc_w1, vc_b1, vc_w2, vc_b2,
           gate_w, gate_b, Wo):
    x = inp[0]
    q, k, v, g = _proj(x, Wq, Wk, Wv, norm_w, gate_w, gate_b)

    def to_blocks(t):
        return (t.reshape(W, BS, H, DH).transpose(2, 0, 1, 3)
                .reshape(H * W, HID))

    ck = _compress(to_blocks(k), jnp.repeat(k_pos.reshape(H, HID), W, axis=0),
                   kc_w1, kc_b1, kc_w2, kc_b2)
    cv = _compress(to_blocks(v), jnp.repeat(v_pos.reshape(H, HID), W, axis=0),
                   vc_w1, vc_b1, vc_w2, vc_b2)
    ckf = jnp.concatenate([mem_kv[0], ck.reshape(H, W, DH)], axis=1)
    cvf = jnp.concatenate([mem_kv[1], cv.reshape(H, W, DH)], axis=1)
    pad = ((0, 0), (0, WP - W - NMEM), (0, 0))
    ckf = jnp.pad(ckf, pad)
    cvf = jnp.pad(cvf, pad)

    pos = jnp.arange(N, dtype=jnp.float32)
    inv = 1.0 / (10000.0 ** (jnp.arange(0, DH, 2, dtype=jnp.float32) / DH))
    f = pos[:, None] * inv[None, :]
    emb = jnp.concatenate([f, f], axis=1)
    cos = jnp.cos(emb)
    sin = jnp.sin(emb)

    def to_heads(t):
        return t.reshape(N, H, DH).transpose(1, 0, 2)

    qh, kh, vh = to_heads(q), to_heads(k), to_heads(v)
    cout, selm = _coarse(qh, ckf, cvf)
    fout = _fine(qh, kh, vh, selm, cos, sin)

    def from_heads(t):
        return t.transpose(1, 0, 2).reshape(N, DIM)

    out = _final(from_heads(cout), from_heads(fout),
                 g[:, 0::2], g[:, 1::2], Wo)
    return out[None]


# speedup vs baseline: 1.1702x; 1.1215x over previous
import jax
import jax.numpy as jnp
from jax.experimental import pallas as pl
from jax.experimental.pallas import tpu as pltpu

B, N, DIM = 1, 2048, 1024
H, DH, BS = 16, 64, 64
W = N // BS
SEL = 8
NMEM = 1
HID = BS * DH
SCALE = DH ** -0.5
NEG = -1e30
WP = 40
QT = 512


_PRECISION = jax.lax.Precision.DEFAULT


def _f32dot(a, b):
    return jnp.dot(a, b, preferred_element_type=jnp.float32,
                   precision=_PRECISION)


def _cdot(a, b):
    return jax.lax.dot_general(a, b, (((1,), (1,)), ((), ())),
                               preferred_element_type=jnp.float32,
                               precision=_PRECISION)


def _proj_kernel(x_ref, wq_ref, wk_ref, wv_ref, nw_ref, gw_ref, gb_ref,
                 q_ref, k_ref, v_ref, g_ref):
    x = x_ref[...]
    q_ref[...] = _f32dot(x, wq_ref[...])
    k_ref[...] = _f32dot(x, wk_ref[...])
    v_ref[...] = _f32dot(x, wv_ref[...])
    ms = jnp.mean(x * x, axis=1, keepdims=True)
    xn = x * jax.lax.rsqrt(ms + 1e-6) * nw_ref[...]
    g_ref[...] = jax.nn.sigmoid(_f32dot(xn, gw_ref[...]) + gb_ref[...])


def _proj(x, Wq, Wk, Wv, norm_w, gate_w, gate_b):
    grid = (N // QT,)
    row = pl.BlockSpec((QT, DIM), lambda i: (i, 0))
    full = pl.BlockSpec((DIM, DIM), lambda i: (0, 0))
    return pl.pallas_call(
        _proj_kernel,
        grid=grid,
        in_specs=[row, full, full, full,
                  pl.BlockSpec((1, DIM), lambda i: (0, 0)),
                  pl.BlockSpec((DIM, 2 * H), lambda i: (0, 0)),
                  pl.BlockSpec((1, 2 * H), lambda i: (0, 0))],
        out_specs=[row, row, row,
                   pl.BlockSpec((QT, 2 * H), lambda i: (i, 0))],
        out_shape=[jax.ShapeDtypeStruct((N, DIM), jnp.float32)] * 3 +
                  [jax.ShapeDtypeStruct((N, 2 * H), jnp.float32)],
    )(x, Wq, Wk, Wv, norm_w[None, :], gate_w, gate_b[None, :])


def _compress_kernel(a_ref, p_ref, w1_ref, b1_ref, w2_ref, b2_ref, out_ref):
    j = pl.program_id(0)
    a = a_ref[...] + p_ref[...]
    h1 = jax.nn.relu(_f32dot(a, w1_ref[...]) + b1_ref[...])

    @pl.when(j == 0)
    def _():
        out_ref[...] = jnp.broadcast_to(b2_ref[...], (H * W, DH))

    out_ref[...] += _f32dot(h1, w2_ref[...])


def _compress(a, p, w1, b1, w2, b2, bn=512):
    grid = (HID // bn,)
    return pl.pallas_call(
        _compress_kernel,
        grid=grid,
        in_specs=[pl.BlockSpec((H * W, HID), lambda j: (0, 0)),
                  pl.BlockSpec((H * W, HID), lambda j: (0, 0)),
                  pl.BlockSpec((HID, bn), lambda j: (0, j)),
                  pl.BlockSpec((1, bn), lambda j: (0, j)),
                  pl.BlockSpec((bn, DH), lambda j: (j, 0)),
                  pl.BlockSpec((1, DH), lambda j: (0, 0))],
        out_specs=pl.BlockSpec((H * W, DH), lambda j: (0, 0)),
        out_shape=jax.ShapeDtypeStruct((H * W, DH), jnp.float32),
    )(a, p, w1, b1[None, :], w2, b2[None, :])


def _rope(x, cos, sin):
    x1 = x[:, :DH // 2]
    x2 = x[:, DH // 2:]
    rot = jnp.concatenate([-x2, x1], axis=1)
    return x * cos + rot * sin


KC = 512
NKC = N // KC


def _coarse_kernel(q_ref, ckf_ref, cvf_ref, cout_ref, selm_ref):
    q = q_ref[0]
    sc = _cdot(ckf_ref[0], q) * SCALE
    c = jax.lax.broadcasted_iota(jnp.int32, (WP, N), 0)
    iq = jax.lax.broadcasted_iota(jnp.int32, (WP, N), 1)
    valid = (c == 0) | ((c <= W) & (iq >= c * BS - 1))
    sc = jnp.where(valid, sc, NEG)
    mc = jnp.max(sc, axis=0, keepdims=True)
    ec = jnp.exp(sc - mc)
    attnc = ec / jnp.sum(ec, axis=0, keepdims=True)
    cout_ref[0] = jax.lax.dot_general(
        attnc, cvf_ref[0], (((0,), (0,)), ((), ())),
        preferred_element_type=jnp.float32, precision=_PRECISION)
    blkrow = (c >= 1) & (c <= W)
    imp = jnp.where(blkrow, attnc, -1.0)
    vt = imp
    thr = vt
    for _ in range(SEL):
        thr = jnp.max(vt, axis=0, keepdims=True)
        vt = jnp.where(vt == thr, -1.0, vt)
    selm_ref[0] = (imp >= thr).astype(jnp.float32)


def _coarse(q, ckf, cvf):
    head = pl.BlockSpec((1, N, DH), lambda h: (h, 0, 0))
    cf = pl.BlockSpec((1, WP, DH), lambda h: (h, 0, 0))
    return pl.pallas_call(
        _coarse_kernel,
        grid=(H,),
        in_specs=[head, cf, cf],
        out_specs=[head, pl.BlockSpec((1, WP, N), lambda h: (h, 0, 0))],
        out_shape=[jax.ShapeDtypeStruct((H, N, DH), jnp.float32),
                   jax.ShapeDtypeStruct((H, WP, N), jnp.float32)],
    )(q, ckf, cvf)


def _fine_kernel(q_ref, k_ref, v_ref, sel_ref, cos_ref, sin_ref,
                 fout_ref, fk_ref):
    qt = pl.program_id(1)

    @pl.when(qt == 0)
    def _():
        fk_ref[...] = _rope(k_ref[0], cos_ref[...], sin_ref[...])

    last = qt * QT // KC
    cs = cos_ref[pl.ds(qt * QT, QT), :]
    sn = sin_ref[pl.ds(qt * QT, QT), :]
    fq = _rope(q_ref[0], cs, sn) * SCALE
    selT = sel_ref[0]

    crow = jax.lax.broadcasted_iota(jnp.int32, (WP, KC), 0)
    pcol = jax.lax.broadcasted_iota(jnp.int32, (WP, KC), 1) // BS + NMEM

    def selmask(c):
        expand = (crow == pcol + c * (KC // BS)).astype(jnp.float32)
        return jax.lax.dot_general(
            selT, expand, (((0,), (0,)), ((), ())),
            preferred_element_type=jnp.float32, precision=_PRECISION)

    def step(m, l, acc, c, maskf, s):
        m_new = jnp.maximum(m, jnp.max(s, axis=1, keepdims=True))
        e = jnp.exp(s - m_new) * maskf
        corr = jnp.exp(m - m_new)
        l_new = l * corr + jnp.sum(e, axis=1, keepdims=True)
        acc_new = acc * corr + _f32dot(e, v_ref[0, pl.ds(c * KC, KC), :])
        return m_new, l_new, acc_new

    def interior(cc, carry):
        m, l, acc = carry
        s = _cdot(fq, fk_ref[pl.ds(cc * KC, KC), :])
        return step(m, l, acc, cc, selmask(cc), s)

    m0 = jnp.full((QT, 1), NEG, jnp.float32)
    l0 = jnp.zeros((QT, 1), jnp.float32)
    a0 = jnp.zeros((QT, DH), jnp.float32)
    m, l, acc = jax.lax.fori_loop(0, last, interior, (m0, l0, a0))

    s = _cdot(fq, fk_ref[pl.ds(last * KC, KC), :])
    i = qt * QT + jax.lax.broadcasted_iota(jnp.int32, (QT, KC), 0)
    p = last * KC + jax.lax.broadcasted_iota(jnp.int32, (QT, KC), 1)
    own = ((p // BS) == (i // BS)).astype(jnp.float32)
    causal = (p <= i).astype(jnp.float32)
    sel = selmask(last)
    maskf = causal * (sel + own - sel * own)
    m, l, acc = step(m, l, acc, last, maskf, s)
    fout_ref[0] = acc / l


def _fine(q, k, v, selm, cos, sin):
    qtile = pl.BlockSpec((1, QT, DH), lambda h, t: (h, t, 0))
    head = pl.BlockSpec((1, N, DH), lambda h, t: (h, 0, 0))
    full = pl.BlockSpec((N, DH), lambda h, t: (0, 0))
    return pl.pallas_call(
        _fine_kernel,
        grid=(H, N // QT),
        in_specs=[qtile, head, head,
                  pl.BlockSpec((1, WP, QT), lambda h, t: (h, 0, t)),
                  full, full],
        out_specs=qtile,
        out_shape=jax.ShapeDtypeStruct((H, N, DH), jnp.float32),
        scratch_shapes=[pltpu.VMEM((N, DH), jnp.float32)],
    )(q, k, v, selm, cos, sin)


def _final_kernel(c_ref, f_ref, gc_ref, gf_ref, wo_ref, out_ref):
    hrow = jax.lax.broadcasted_iota(jnp.int32, (H, DIM), 0)
    dcol = jax.lax.broadcasted_iota(jnp.int32, (H, DIM), 1)
    expand = (hrow == dcol // DH).astype(jnp.float32)
    gc = jnp.dot(gc_ref[...], expand, preferred_element_type=jnp.float32,
                 precision=jax.lax.Precision.HIGHEST)
    gf = jnp.dot(gf_ref[...], expand, preferred_element_type=jnp.float32,
                 precision=jax.lax.Precision.HIGHEST)
    merged = gc * c_ref[...] + gf * f_ref[...]
    out_ref[...] = _f32dot(merged, wo_ref[...])


def _final(c, f, gc, gf, Wo):
    row = pl.BlockSpec((QT, DIM), lambda i: (i, 0))
    return pl.pallas_call(
        _final_kernel,
        grid=(N // QT,),
        in_specs=[row, row,
                  pl.BlockSpec((QT, H), lambda i: (i, 0)),
                  pl.BlockSpec((QT, H), lambda i: (i, 0)),
                  pl.BlockSpec((DIM, DIM), lambda i: (0, 0))],
        out_specs=row,
        out_shape=jax.ShapeDtypeStruct((N, DIM), jnp.float32),
    )(c, f, gc, gf, Wo)


def kernel(inp, Wq, Wk, Wv, norm_w, mem_kv, k_pos, v_pos,
           kc_w1, kc_b1, kc_w2, kc_b2, vc_w1, vc_b1, vc_w2, vc_b2,
           gate_w, gate_b, Wo):
    x = inp[0]
    q, k, v, g = _proj(x, Wq, Wk, Wv, norm_w, gate_w, gate_b)

    def to_blocks(t):
        return (t.reshape(W, BS, H, DH).transpose(2, 0, 1, 3)
                .reshape(H * W, HID))

    ck = _compress(to_blocks(k), jnp.repeat(k_pos.reshape(H, HID), W, axis=0),
                   kc_w1, kc_b1, kc_w2, kc_b2)
    cv = _compress(to_blocks(v), jnp.repeat(v_pos.reshape(H, HID), W, axis=0),
                   vc_w1, vc_b1, vc_w2, vc_b2)
    ckf = jnp.concatenate([mem_kv[0], ck.reshape(H, W, DH)], axis=1)
    cvf = jnp.concatenate([mem_kv[1], cv.reshape(H, W, DH)], axis=1)
    pad = ((0, 0), (0, WP - W - NMEM), (0, 0))
    ckf = jnp.pad(ckf, pad)
    cvf = jnp.pad(cvf, pad)

    pos = jnp.arange(N, dtype=jnp.float32)
    inv = 1.0 / (10000.0 ** (jnp.arange(0, DH, 2, dtype=jnp.float32) / DH))
    f = pos[:, None] * inv[None, :]
    emb = jnp.concatenate([f, f], axis=1)
    cos = jnp.cos(emb)
    sin = jnp.sin(emb)

    def to_heads(t):
        return t.reshape(N, H, DH).transpose(1, 0, 2)

    qh, kh, vh = to_heads(q), to_heads(k), to_heads(v)
    cout, selm = _coarse(qh, ckf, cvf)
    fout = _fine(qh, kh, vh, selm, cos, sin)

    def from_heads(t):
        return t.transpose(1, 0, 2).reshape(N, DIM)

    out = _final(from_heads(cout), from_heads(fout),
                 g[:, 0::2], g[:, 1::2], Wo)
    return out[None]


# speedup vs baseline: 1.2301x; 1.0511x over previous
import jax
import jax.numpy as jnp
from jax.experimental import pallas as pl
from jax.experimental.pallas import tpu as pltpu

B, N, DIM = 1, 2048, 1024
H, DH, BS = 16, 64, 64
W = N // BS
SEL = 8
NMEM = 1
HID = BS * DH
SCALE = DH ** -0.5
NEG = -1e30
WP = 40
QT = 512


_PRECISION = jax.lax.Precision.DEFAULT


def _f32dot(a, b):
    return jnp.dot(a, b, preferred_element_type=jnp.float32,
                   precision=_PRECISION)


def _cdot(a, b):
    return jax.lax.dot_general(a, b, (((1,), (1,)), ((), ())),
                               preferred_element_type=jnp.float32,
                               precision=_PRECISION)


def _proj_kernel(x_ref, wq_ref, wk_ref, wv_ref, nw_ref, gw_ref, gb_ref,
                 q_ref, k_ref, v_ref, g_ref):
    x = x_ref[...]
    q_ref[...] = _f32dot(x, wq_ref[...])
    k_ref[...] = _f32dot(x, wk_ref[...])
    v_ref[...] = _f32dot(x, wv_ref[...])
    ms = jnp.mean(x * x, axis=1, keepdims=True)
    xn = x * jax.lax.rsqrt(ms + 1e-6) * nw_ref[...]
    g_ref[...] = jax.nn.sigmoid(_f32dot(xn, gw_ref[...]) + gb_ref[...])


def _proj(x, Wq, Wk, Wv, norm_w, gate_w, gate_b):
    grid = (N // QT,)
    row = pl.BlockSpec((QT, DIM), lambda i: (i, 0))
    full = pl.BlockSpec((DIM, DIM), lambda i: (0, 0))
    return pl.pallas_call(
        _proj_kernel,
        grid=grid,
        in_specs=[row, full, full, full,
                  pl.BlockSpec((1, DIM), lambda i: (0, 0)),
                  pl.BlockSpec((DIM, 2 * H), lambda i: (0, 0)),
                  pl.BlockSpec((1, 2 * H), lambda i: (0, 0))],
        out_specs=[row, row, row,
                   pl.BlockSpec((QT, 2 * H), lambda i: (i, 0))],
        out_shape=[jax.ShapeDtypeStruct((N, DIM), jnp.float32)] * 3 +
                  [jax.ShapeDtypeStruct((N, 2 * H), jnp.float32)],
    )(x, Wq, Wk, Wv, norm_w[None, :], gate_w, gate_b[None, :])


def _compress_kernel(a_ref, p_ref, w1_ref, b1_ref, w2_ref, b2_ref, out_ref):
    j = pl.program_id(0)
    a = a_ref[...] + p_ref[...]
    h1 = jax.nn.relu(_f32dot(a, w1_ref[...]) + b1_ref[...])

    @pl.when(j == 0)
    def _():
        out_ref[...] = jnp.broadcast_to(b2_ref[...], (H * W, DH))

    out_ref[...] += _f32dot(h1, w2_ref[...])


def _compress(a, p, w1, b1, w2, b2, bn=512):
    grid = (HID // bn,)
    return pl.pallas_call(
        _compress_kernel,
        grid=grid,
        in_specs=[pl.BlockSpec((H * W, HID), lambda j: (0, 0)),
                  pl.BlockSpec((H * W, HID), lambda j: (0, 0)),
                  pl.BlockSpec((HID, bn), lambda j: (0, j)),
                  pl.BlockSpec((1, bn), lambda j: (0, j)),
                  pl.BlockSpec((bn, DH), lambda j: (j, 0)),
                  pl.BlockSpec((1, DH), lambda j: (0, 0))],
        out_specs=pl.BlockSpec((H * W, DH), lambda j: (0, 0)),
        out_shape=jax.ShapeDtypeStruct((H * W, DH), jnp.float32),
    )(a, p, w1, b1[None, :], w2, b2[None, :])


def _rope(x, cos, sin):
    x1 = x[:, :DH // 2]
    x2 = x[:, DH // 2:]
    rot = jnp.concatenate([-x2, x1], axis=1)
    return x * cos + rot * sin


KC = 512
NKC = N // KC


def _coarse_kernel(q_ref, ckf_ref, cvf_ref, cout_ref, selm_ref):
    q = q_ref[0]
    sc = _cdot(ckf_ref[0], q) * SCALE
    c = jax.lax.broadcasted_iota(jnp.int32, (WP, N), 0)
    iq = jax.lax.broadcasted_iota(jnp.int32, (WP, N), 1)
    valid = (c == 0) | ((c <= W) & (iq >= c * BS - 1))
    sc = jnp.where(valid, sc, NEG)
    mc = jnp.max(sc, axis=0, keepdims=True)
    ec = jnp.exp(sc - mc)
    attnc = ec / jnp.sum(ec, axis=0, keepdims=True)
    cout_ref[0] = jax.lax.dot_general(
        attnc, cvf_ref[0], (((0,), (0,)), ((), ())),
        preferred_element_type=jnp.float32, precision=_PRECISION)
    blkrow = (c >= 1) & (c <= W)
    imp = jnp.where(blkrow, attnc, -1.0)
    vt = imp
    thr = vt
    for _ in range(SEL):
        thr = jnp.max(vt, axis=0, keepdims=True)
        vt = jnp.where(vt == thr, -1.0, vt)
    selm_ref[0] = (imp >= thr).astype(jnp.float32)


def _coarse(q, ckf, cvf):
    head = pl.BlockSpec((1, N, DH), lambda h: (h, 0, 0))
    cf = pl.BlockSpec((1, WP, DH), lambda h: (h, 0, 0))
    return pl.pallas_call(
        _coarse_kernel,
        grid=(H,),
        in_specs=[head, cf, cf],
        out_specs=[head, pl.BlockSpec((1, WP, N), lambda h: (h, 0, 0))],
        out_shape=[jax.ShapeDtypeStruct((H, N, DH), jnp.float32),
                   jax.ShapeDtypeStruct((H, WP, N), jnp.float32)],
    )(q, ckf, cvf)


def _fine_kernel(q_ref, k_ref, v_ref, sel_ref, cos_ref, sin_ref,
                 fout_ref, fk_ref):
    qt = pl.program_id(1)

    @pl.when(qt == 0)
    def _():
        fk_ref[...] = _rope(k_ref[0], cos_ref[...], sin_ref[...])

    last = qt * QT // KC
    cs = cos_ref[pl.ds(qt * QT, QT), :]
    sn = sin_ref[pl.ds(qt * QT, QT), :]
    fq = _rope(q_ref[0], cs, sn) * SCALE
    selT = sel_ref[0]

    crow = jax.lax.broadcasted_iota(jnp.int32, (WP, KC), 0)
    pcol = jax.lax.broadcasted_iota(jnp.int32, (WP, KC), 1) // BS + NMEM

    def selmask(c):
        expand = (crow == pcol + c * (KC // BS)).astype(jnp.float32)
        return jax.lax.dot_general(
            selT, expand, (((0,), (0,)), ((), ())),
            preferred_element_type=jnp.float32, precision=_PRECISION)

    def step(m, l, acc, c, maskf, s):
        m_new = jnp.maximum(m, jnp.max(s, axis=1, keepdims=True))
        e = jnp.exp(s - m_new) * maskf
        corr = jnp.exp(m - m_new)
        l_new = l * corr + jnp.sum(e, axis=1, keepdims=True)
        acc_new = acc * corr + _f32dot(e, v_ref[0, pl.ds(c * KC, KC), :])
        return m_new, l_new, acc_new

    def interior(cc, carry):
        m, l, acc = carry
        s = _cdot(fq, fk_ref[pl.ds(cc * KC, KC), :])
        return step(m, l, acc, cc, selmask(cc), s)

    m0 = jnp.full((QT, 1), NEG, jnp.float32)
    l0 = jnp.zeros((QT, 1), jnp.float32)
    a0 = jnp.zeros((QT, DH), jnp.float32)
    m, l, acc = jax.lax.fori_loop(0, last, interior, (m0, l0, a0))

    s = _cdot(fq, fk_ref[pl.ds(last * KC, KC), :])
    i = qt * QT + jax.lax.broadcasted_iota(jnp.int32, (QT, KC), 0)
    p = last * KC + jax.lax.broadcasted_iota(jnp.int32, (QT, KC), 1)
    own = ((p // BS) == (i // BS)).astype(jnp.float32)
    causal = (p <= i).astype(jnp.float32)
    sel = selmask(last)
    maskf = causal * (sel + own - sel * own)
    m, l, acc = step(m, l, acc, last, maskf, s)
    fout_ref[0] = acc / l


def _fine(q, k, v, selm, cos, sin):
    qtile = pl.BlockSpec((1, QT, DH), lambda h, t: (h, t, 0))
    head = pl.BlockSpec((1, N, DH), lambda h, t: (h, 0, 0))
    full = pl.BlockSpec((N, DH), lambda h, t: (0, 0))
    return pl.pallas_call(
        _fine_kernel,
        grid=(H, N // QT),
        in_specs=[qtile, head, head,
                  pl.BlockSpec((1, WP, QT), lambda h, t: (h, 0, t)),
                  full, full],
        out_specs=qtile,
        out_shape=jax.ShapeDtypeStruct((H, N, DH), jnp.float32),
        scratch_shapes=[pltpu.VMEM((N, DH), jnp.float32)],
    )(q, k, v, selm, cos, sin)


def _final_kernel(c_ref, f_ref, gc_ref, gf_ref, wo_ref, out_ref):
    acc = jnp.zeros((QT, DIM), jnp.float32)
    for h in range(H):
        mh = (gc_ref[:, h:h + 1] * c_ref[h] +
              gf_ref[:, h:h + 1] * f_ref[h])
        acc = acc + _f32dot(mh, wo_ref[h * DH:(h + 1) * DH, :])
    out_ref[...] = acc


def _final(c, f, gc, gf, Wo):
    row = pl.BlockSpec((QT, DIM), lambda i: (i, 0))
    return pl.pallas_call(
        _final_kernel,
        grid=(N // QT,),
        in_specs=[pl.BlockSpec((H, QT, DH), lambda i: (0, i, 0)),
                  pl.BlockSpec((H, QT, DH), lambda i: (0, i, 0)),
                  pl.BlockSpec((QT, H), lambda i: (i, 0)),
                  pl.BlockSpec((QT, H), lambda i: (i, 0)),
                  pl.BlockSpec((DIM, DIM), lambda i: (0, 0))],
        out_specs=row,
        out_shape=jax.ShapeDtypeStruct((N, DIM), jnp.float32),
    )(c, f, gc, gf, Wo)


def kernel(inp, Wq, Wk, Wv, norm_w, mem_kv, k_pos, v_pos,
           kc_w1, kc_b1, kc_w2, kc_b2, vc_w1, vc_b1, vc_w2, vc_b2,
           gate_w, gate_b, Wo):
    x = inp[0]
    q, k, v, g = _proj(x, Wq, Wk, Wv, norm_w, gate_w, gate_b)

    def to_blocks(t):
        return (t.reshape(W, BS, H, DH).transpose(2, 0, 1, 3)
                .reshape(H * W, HID))

    ck = _compress(to_blocks(k), jnp.repeat(k_pos.reshape(H, HID), W, axis=0),
                   kc_w1, kc_b1, kc_w2, kc_b2)
    cv = _compress(to_blocks(v), jnp.repeat(v_pos.reshape(H, HID), W, axis=0),
                   vc_w1, vc_b1, vc_w2, vc_b2)
    ckf = jnp.concatenate([mem_kv[0], ck.reshape(H, W, DH)], axis=1)
    cvf = jnp.concatenate([mem_kv[1], cv.reshape(H, W, DH)], axis=1)
    pad = ((0, 0), (0, WP - W - NMEM), (0, 0))
    ckf = jnp.pad(ckf, pad)
    cvf = jnp.pad(cvf, pad)

    pos = jnp.arange(N, dtype=jnp.float32)
    inv = 1.0 / (10000.0 ** (jnp.arange(0, DH, 2, dtype=jnp.float32) / DH))
    f = pos[:, None] * inv[None, :]
    emb = jnp.concatenate([f, f], axis=1)
    cos = jnp.cos(emb)
    sin = jnp.sin(emb)

    def to_heads(t):
        return t.reshape(N, H, DH).transpose(1, 0, 2)

    qh, kh, vh = to_heads(q), to_heads(k), to_heads(v)
    cout, selm = _coarse(qh, ckf, cvf)
    fout = _fine(qh, kh, vh, selm, cos, sin)

    out = _final(cout, fout, g[:, 0::2], g[:, 1::2], Wo)
    return out[None]


# speedup vs baseline: 1.2321x; 1.0017x over previous
import jax
import jax.numpy as jnp
from jax.experimental import pallas as pl
from jax.experimental.pallas import tpu as pltpu

B, N, DIM = 1, 2048, 1024
H, DH, BS = 16, 64, 64
W = N // BS
SEL = 8
NMEM = 1
HID = BS * DH
SCALE = DH ** -0.5
NEG = -1e30
WP = 40
QT = 512


_PRECISION = jax.lax.Precision.DEFAULT


def _f32dot(a, b):
    return jnp.dot(a, b, preferred_element_type=jnp.float32,
                   precision=_PRECISION)


def _cdot(a, b):
    return jax.lax.dot_general(a, b, (((1,), (1,)), ((), ())),
                               preferred_element_type=jnp.float32,
                               precision=_PRECISION)


def _proj_kernel(x_ref, wq_ref, wk_ref, wv_ref, nw_ref, gw_ref, gb_ref,
                 q_ref, k_ref, v_ref, g_ref):
    x = x_ref[...]
    q_ref[...] = _f32dot(x, wq_ref[...])
    k_ref[...] = _f32dot(x, wk_ref[...])
    v_ref[...] = _f32dot(x, wv_ref[...])
    ms = jnp.mean(x * x, axis=1, keepdims=True)
    xn = x * jax.lax.rsqrt(ms + 1e-6) * nw_ref[...]
    g_ref[...] = jax.nn.sigmoid(_f32dot(xn, gw_ref[...]) + gb_ref[...])


def _proj(x, Wq, Wk, Wv, norm_w, gate_w, gate_b):
    grid = (N // QT,)
    row = pl.BlockSpec((QT, DIM), lambda i: (i, 0))
    full = pl.BlockSpec((DIM, DIM), lambda i: (0, 0))
    return pl.pallas_call(
        _proj_kernel,
        grid=grid,
        in_specs=[row, full, full, full,
                  pl.BlockSpec((1, DIM), lambda i: (0, 0)),
                  pl.BlockSpec((DIM, 2 * H), lambda i: (0, 0)),
                  pl.BlockSpec((1, 2 * H), lambda i: (0, 0))],
        out_specs=[row, row, row,
                   pl.BlockSpec((QT, 2 * H), lambda i: (i, 0))],
        out_shape=[jax.ShapeDtypeStruct((N, DIM), jnp.float32)] * 3 +
                  [jax.ShapeDtypeStruct((N, 2 * H), jnp.float32)],
    )(x, Wq, Wk, Wv, norm_w[None, :], gate_w, gate_b[None, :])


def _compress_kernel(a_ref, p_ref, w1_ref, b1_ref, w2_ref, b2_ref, out_ref):
    j = pl.program_id(0)
    a = a_ref[...] + p_ref[...]
    h1 = jax.nn.relu(_f32dot(a, w1_ref[...]) + b1_ref[...])

    @pl.when(j == 0)
    def _():
        out_ref[...] = jnp.broadcast_to(b2_ref[...], (H * W, DH))

    out_ref[...] += _f32dot(h1, w2_ref[...])


def _compress(a, p, w1, b1, w2, b2, bn=512):
    grid = (HID // bn,)
    return pl.pallas_call(
        _compress_kernel,
        grid=grid,
        in_specs=[pl.BlockSpec((H * W, HID), lambda j: (0, 0)),
                  pl.BlockSpec((H * W, HID), lambda j: (0, 0)),
                  pl.BlockSpec((HID, bn), lambda j: (0, j)),
                  pl.BlockSpec((1, bn), lambda j: (0, j)),
                  pl.BlockSpec((bn, DH), lambda j: (j, 0)),
                  pl.BlockSpec((1, DH), lambda j: (0, 0))],
        out_specs=pl.BlockSpec((H * W, DH), lambda j: (0, 0)),
        out_shape=jax.ShapeDtypeStruct((H * W, DH), jnp.float32),
    )(a, p, w1, b1[None, :], w2, b2[None, :])


def _rope(x, cos, sin):
    x1 = x[:, :DH // 2]
    x2 = x[:, DH // 2:]
    rot = jnp.concatenate([-x2, x1], axis=1)
    return x * cos + rot * sin


KC = 512


def _coarse_kernel(q_ref, ckf_ref, cvf_ref, cout_ref, selm_ref):
    q = q_ref[0]
    sc = _cdot(ckf_ref[0], q) * SCALE
    c = jax.lax.broadcasted_iota(jnp.int32, (WP, N), 0)
    iq = jax.lax.broadcasted_iota(jnp.int32, (WP, N), 1)
    valid = (c == 0) | ((c <= W) & (iq >= c * BS - 1))
    sc = jnp.where(valid, sc, NEG)
    mc = jnp.max(sc, axis=0, keepdims=True)
    ec = jnp.exp(sc - mc)
    attnc = ec / jnp.sum(ec, axis=0, keepdims=True)
    cout_ref[0] = jax.lax.dot_general(
        attnc, cvf_ref[0], (((0,), (0,)), ((), ())),
        preferred_element_type=jnp.float32, precision=_PRECISION)
    blkrow = (c >= 1) & (c <= W)
    imp = jnp.where(blkrow, attnc, -1.0)
    vt = imp
    thr = vt
    for _ in range(SEL):
        thr = jnp.max(vt, axis=0, keepdims=True)
        vt = jnp.where(vt == thr, -1.0, vt)
    selm_ref[0] = (imp >= thr).astype(jnp.float32)


def _coarse(q, ckf, cvf):
    head = pl.BlockSpec((1, N, DH), lambda h: (h, 0, 0))
    cf = pl.BlockSpec((1, WP, DH), lambda h: (h, 0, 0))
    return pl.pallas_call(
        _coarse_kernel,
        grid=(H,),
        in_specs=[head, cf, cf],
        out_specs=[head, pl.BlockSpec((1, WP, N), lambda h: (h, 0, 0))],
        out_shape=[jax.ShapeDtypeStruct((H, N, DH), jnp.float32),
                   jax.ShapeDtypeStruct((H, WP, N), jnp.float32)],
    )(q, ckf, cvf)


def _fine_kernel(q_ref, k_ref, v_ref, sel_ref, cos_ref, sin_ref,
                 fout_ref, fk_ref):
    qt = pl.program_id(1)

    @pl.when(qt == 0)
    def _():
        fk_ref[...] = _rope(k_ref[0], cos_ref[...], sin_ref[...])

    last = qt * QT // KC
    cs = cos_ref[pl.ds(qt * QT, QT), :]
    sn = sin_ref[pl.ds(qt * QT, QT), :]
    fq = _rope(q_ref[0], cs, sn) * SCALE
    selT = sel_ref[0]

    crow = jax.lax.broadcasted_iota(jnp.int32, (WP, KC), 0)
    pcol = jax.lax.broadcasted_iota(jnp.int32, (WP, KC), 1) // BS + NMEM

    def selmask(c):
        expand = (crow == pcol + c * (KC // BS)).astype(jnp.float32)
        return jax.lax.dot_general(
            selT, expand, (((0,), (0,)), ((), ())),
            preferred_element_type=jnp.float32, precision=_PRECISION)

    def step(m, l, acc, c, maskf, s):
        m_new = jnp.maximum(m, jnp.max(s, axis=1, keepdims=True))
        e = jnp.exp(s - m_new) * maskf
        corr = jnp.exp(m - m_new)
        l_new = l * corr + jnp.sum(e, axis=1, keepdims=True)
        acc_new = acc * corr + _f32dot(e, v_ref[0, pl.ds(c * KC, KC), :])
        return m_new, l_new, acc_new

    def interior(cc, carry):
        m, l, acc = carry
        s = _cdot(fq, fk_ref[pl.ds(cc * KC, KC), :])
        return step(m, l, acc, cc, selmask(cc), s)

    m0 = jnp.full((QT, 1), NEG, jnp.float32)
    l0 = jnp.zeros((QT, 1), jnp.float32)
    a0 = jnp.zeros((QT, DH), jnp.float32)
    m, l, acc = jax.lax.fori_loop(0, last, interior, (m0, l0, a0))

    s = _cdot(fq, fk_ref[pl.ds(last * KC, KC), :])
    i = qt * QT + jax.lax.broadcasted_iota(jnp.int32, (QT, KC), 0)
    p = last * KC + jax.lax.broadcasted_iota(jnp.int32, (QT, KC), 1)
    own = ((p // BS) == (i // BS)).astype(jnp.float32)
    causal = (p <= i).astype(jnp.float32)
    sel = selmask(last)
    maskf = causal * (sel + own - sel * own)
    m, l, acc = step(m, l, acc, last, maskf, s)
    fout_ref[0] = acc / l


def _fine(q, k, v, selm, cos, sin):
    qtile = pl.BlockSpec((1, QT, DH), lambda h, t: (h, t, 0))
    head = pl.BlockSpec((1, N, DH), lambda h, t: (h, 0, 0))
    full = pl.BlockSpec((N, DH), lambda h, t: (0, 0))
    return pl.pallas_call(
        _fine_kernel,
        grid=(H, N // QT),
        in_specs=[qtile, head, head,
                  pl.BlockSpec((1, WP, QT), lambda h, t: (h, 0, t)),
                  full, full],
        out_specs=qtile,
        out_shape=jax.ShapeDtypeStruct((H, N, DH), jnp.float32),
        scratch_shapes=[pltpu.VMEM((N, DH), jnp.float32)],
    )(q, k, v, selm, cos, sin)


def _final_kernel(c_ref, f_ref, gc_ref, gf_ref, wo_ref, out_ref):
    acc = jnp.zeros((QT, DIM), jnp.float32)
    for h in range(H):
        mh = (gc_ref[:, h:h + 1] * c_ref[h] +
              gf_ref[:, h:h + 1] * f_ref[h])
        acc = acc + _f32dot(mh, wo_ref[h * DH:(h + 1) * DH, :])
    out_ref[...] = acc


def _final(c, f, gc, gf, Wo):
    row = pl.BlockSpec((QT, DIM), lambda i: (i, 0))
    return pl.pallas_call(
        _final_kernel,
        grid=(N // QT,),
        in_specs=[pl.BlockSpec((H, QT, DH), lambda i: (0, i, 0)),
                  pl.BlockSpec((H, QT, DH), lambda i: (0, i, 0)),
                  pl.BlockSpec((QT, H), lambda i: (i, 0)),
                  pl.BlockSpec((QT, H), lambda i: (i, 0)),
                  pl.BlockSpec((DIM, DIM), lambda i: (0, 0))],
        out_specs=row,
        out_shape=jax.ShapeDtypeStruct((N, DIM), jnp.float32),
    )(c, f, gc, gf, Wo)


def kernel(inp, Wq, Wk, Wv, norm_w, mem_kv, k_pos, v_pos,
           kc_w1, kc_b1, kc_w2, kc_b2, vc_w1, vc_b1, vc_w2, vc_b2,
           gate_w, gate_b, Wo):
    x = inp[0]
    q, k, v, g = _proj(x, Wq, Wk, Wv, norm_w, gate_w, gate_b)

    def to_blocks(t):
        return (t.reshape(W, BS, H, DH).transpose(2, 0, 1, 3)
                .reshape(H * W, HID))

    ck = _compress(to_blocks(k), jnp.repeat(k_pos.reshape(H, HID), W, axis=0),
                   kc_w1, kc_b1, kc_w2, kc_b2)
    cv = _compress(to_blocks(v), jnp.repeat(v_pos.reshape(H, HID), W, axis=0),
                   vc_w1, vc_b1, vc_w2, vc_b2)
    ckf = jnp.concatenate([mem_kv[0], ck.reshape(H, W, DH)], axis=1)
    cvf = jnp.concatenate([mem_kv[1], cv.reshape(H, W, DH)], axis=1)
    pad = ((0, 0), (0, WP - W - NMEM), (0, 0))
    ckf = jnp.pad(ckf, pad)
    cvf = jnp.pad(cvf, pad)

    pos = jnp.arange(N, dtype=jnp.float32)
    inv = 1.0 / (10000.0 ** (jnp.arange(0, DH, 2, dtype=jnp.float32) / DH))
    f = pos[:, None] * inv[None, :]
    emb = jnp.concatenate([f, f], axis=1)
    cos = jnp.cos(emb)
    sin = jnp.sin(emb)

    def to_heads(t):
        return t.reshape(N, H, DH).transpose(1, 0, 2)

    qh, kh, vh = to_heads(q), to_heads(k), to_heads(v)
    cout, selm = _coarse(qh, ckf, cvf)
    fout = _fine(qh, kh, vh, selm, cos, sin)

    out = _final(cout, fout, g[:, 0::2], g[:, 1::2], Wo)
    return out[None]
